# baseline (speedup 1.0000x reference)
import jax
import jax.numpy as jnp
from jax import lax
from jax.experimental import pallas as pl
from jax.experimental.pallas import tpu as pltpu
from jax.experimental.pallas import tpu_sc as plsc

_D = 64
_B = 16384
_NC = 2
_NS = 16
_NW = _NC * _NS
_BW = _B // _NW
_CHUNK = 128
_NCH = _BW // _CHUNK


def _gather_body(table_hbm, idx_hbm, out_hbm, idx_v, rows_v, sem):
    wid = lax.axis_index("s") * _NC + lax.axis_index("c")
    pltpu.sync_copy(idx_hbm.at[wid], idx_v)
    copies = [
        pltpu.async_copy(
            table_hbm.at[idx_v.at[j]],
            rows_v.at[pl.ds(j * _CHUNK, _CHUNK)],
            sem,
        )
        for j in range(_NCH)
    ]
    for c in copies:
        c.wait()
    pltpu.sync_copy(rows_v, out_hbm.at[pl.ds(wid * _BW, _BW)])


_gather = pl.kernel(
    _gather_body,
    mesh=plsc.VectorSubcoreMesh(core_axis_name="c", subcore_axis_name="s"),
    out_type=jax.ShapeDtypeStruct((_B, _D), jnp.float32),
    scratch_types=[
        pltpu.VMEM((_NCH, _CHUNK), jnp.int32),
        pltpu.VMEM((_BW, _D), jnp.float32),
        pltpu.SemaphoreType.DMA,
    ],
    compiler_params=pltpu.CompilerParams(use_tc_tiling_on_sc=False),
)


def kernel(node_vecs, country_idx):
    idx = country_idx.reshape(_NW, _NCH, _CHUNK).astype(jnp.int32)
    return _gather(node_vecs, idx)


# speedup vs baseline: 1.0057x; 1.0057x over previous
import jax
import jax.numpy as jnp
from jax import lax
from jax.experimental import pallas as pl
from jax.experimental.pallas import tpu as pltpu
from jax.experimental.pallas import tpu_sc as plsc

_D = 64
_B = 16384
_NC = 2
_NS = 16
_NW = _NC * _NS
_BW = _B // _NW
_CHUNK = 128
_NCH = _BW // _CHUNK


def _gather_body(table_hbm, idx_hbm, out_hbm, idx_v, rows_v, sem):
    wid = lax.axis_index("s") * _NC + lax.axis_index("c")
    pltpu.sync_copy(idx_hbm.at[wid], idx_v)
    copies = [
        pltpu.async_copy(
            table_hbm.at[idx_v.at[j]],
            rows_v.at[pl.ds(j * _CHUNK, _CHUNK)],
            sem,
        )
        for j in range(_NCH)
    ]
    for c in copies:
        c.wait()
    pltpu.sync_copy(rows_v, out_hbm.at[pl.ds(wid * _BW, _BW)])


_gather = pl.kernel(
    _gather_body,
    mesh=plsc.VectorSubcoreMesh(core_axis_name="c", subcore_axis_name="s"),
    out_type=jax.ShapeDtypeStruct((_B, 2 * _D), jnp.float32),
    scratch_types=[
        pltpu.VMEM((_NCH, _CHUNK), jnp.int32),
        pltpu.VMEM((_BW, 2 * _D), jnp.float32),
        pltpu.SemaphoreType.DMA,
    ],
)


def kernel(node_vecs, country_idx):
    table2 = node_vecs.reshape(500000, 128)
    idx = country_idx.reshape(_B).astype(jnp.int32)
    idxp = (idx >> 1).reshape(_NW, _NCH, _CHUNK)
    pairs = _gather(table2, idxp)
    return pairs[:, :_D]


# speedup vs baseline: 1.0264x; 1.0206x over previous
import jax
import jax.numpy as jnp
from jax import lax
from jax.experimental import pallas as pl
from jax.experimental.pallas import tpu as pltpu
from jax.experimental.pallas import tpu_sc as plsc

_D = 64
_B = 16384
_NC = 2
_NS = 16
_NW = _NC * _NS
_BW = _B // _NW
_BATCH = 32
_NB = _BW // _BATCH


def _gather_body(table_hbm, idx_hbm, out_hbm, idx_v, sem):
    wid = lax.axis_index("s") * _NC + lax.axis_index("c")
    base = wid * _BW
    pltpu.sync_copy(idx_hbm.at[pl.ds(base, _BW)], idx_v)

    def batch(g, carry):
        for h in range(_BATCH // 16):
            vec = idx_v[pl.ds(g * _BATCH + h * 16, 16)]
            for j in range(16):
                i = g * _BATCH + h * 16 + j
                pltpu.async_copy(
                    table_hbm.at[pl.ds(vec[j], 1)],
                    out_hbm.at[pl.ds(base + i, 1)],
                    sem,
                )
        pltpu.make_async_copy(
            table_hbm.at[pl.ds(0, _BATCH)],
            out_hbm.at[pl.ds(base + g * _BATCH, _BATCH)],
            sem,
        ).wait()
        return carry

    lax.fori_loop(0, _NB, batch, 0)


_gather = pl.kernel(
    _gather_body,
    mesh=plsc.VectorSubcoreMesh(core_axis_name="c", subcore_axis_name="s"),
    out_type=jax.ShapeDtypeStruct((_B, _D), jnp.float32),
    scratch_types=[
        pltpu.VMEM((_BW,), jnp.int32),
        pltpu.SemaphoreType.DMA,
    ],
)


def kernel(node_vecs, country_idx):
    idx = country_idx.reshape(_B).astype(jnp.int32)
    return _gather(node_vecs, idx)
